# Initial kernel scaffold; baseline (speedup 1.0000x reference)
import jax
import jax.numpy as jnp
from jax.experimental import pallas as pl


def kernel(x, edge_index, edge_attr, W_node, b_node, W_msg, b_msg):
    raise NotImplementedError("write your pallas kernel here")


# speedup vs baseline: 2.6043x; 2.6043x over previous
import functools

import jax
import jax.numpy as jnp
from jax import lax
from jax.experimental import pallas as pl
from jax.experimental.pallas import tpu as pltpu
from jax.experimental.pallas import tpu_sc as plsc

N = 10000
D = 256
DE = 16
E = 160000

NC = 2
NS = 16
NW = NC * NS
EPW = 5120
E_PAD = NW * EPW
CH = 128
VPW = 2 * EPW
NCH = VPW // CH
N_A = 10240
AGF = 2 * N_A
RPF = AGF // NS


def _proj_body(x_ref, w_ref, b_ref, o_ref):
    o_ref[...] = jnp.dot(x_ref[...], w_ref[...],
                         preferred_element_type=jnp.float32) + b_ref[...]


def _node_proj(x, wc, bc):
    return pl.pallas_call(
        _proj_body,
        grid=(5,),
        in_specs=[
            pl.BlockSpec((2000, D), lambda i: (i, 0)),
            pl.BlockSpec((D, 4), lambda i: (0, 0)),
            pl.BlockSpec((1, 4), lambda i: (0, 0)),
        ],
        out_specs=pl.BlockSpec((2000, 4), lambda i: (i, 0)),
        out_shape=jax.ShapeDtypeStruct((N, 4), jnp.float32),
    )(x, wc, bc)


def _edge_proj(ea, we, bm):
    return pl.pallas_call(
        _proj_body,
        grid=(10,),
        in_specs=[
            pl.BlockSpec((E_PAD // 10, DE), lambda i: (i, 0)),
            pl.BlockSpec((DE, 2), lambda i: (0, 0)),
            pl.BlockSpec((1, 2), lambda i: (0, 0)),
        ],
        out_specs=pl.BlockSpec((E_PAD // 10, 2), lambda i: (i, 0)),
        out_shape=jax.ShapeDtypeStruct((E_PAD, 2), jnp.float32),
    )(ea, we, bm)


def _finish_body(nf_ref, a0_ref, a1_ref, o_ref):
    z = nf_ref[...] + a0_ref[...] + a1_ref[...]
    m = jnp.max(z, axis=1, keepdims=True)
    s = jnp.sum(jnp.exp(z - m), axis=1, keepdims=True)
    o_ref[...] = z - m - jnp.log(s)


def _finish(nf, a0, a1):
    return pl.pallas_call(
        _finish_body,
        grid=(5,),
        in_specs=[pl.BlockSpec((2000, 2), lambda i: (i, 0))] * 3,
        out_specs=pl.BlockSpec((2000, 2), lambda i: (i, 0)),
        out_shape=jax.ShapeDtypeStruct((N, 2), jnp.float32),
    )(nf, a0, a1)


def _sc_body(msg_hbm, src_hbm, dst_hbm, el_hbm, z_hbm, out_hbm,
             table_v, src_v, dst_v, vals_v, agg_sh):
    cid = lax.axis_index("c")
    sid = lax.axis_index("s")
    wid = cid * NS + sid

    pltpu.sync_copy(z_hbm.at[pl.ds(sid * RPF, RPF)],
                    agg_sh.at[pl.ds(sid * RPF, RPF)])
    pltpu.sync_copy(msg_hbm, table_v)
    pltpu.sync_copy(src_hbm.at[wid], src_v)
    pltpu.sync_copy(dst_hbm.at[wid], dst_v)
    pltpu.sync_copy(el_hbm.at[wid], vals_v)
    plsc.subcore_barrier()

    iota16 = lax.iota(jnp.int32, 16)

    def step(i, carry):
        f16 = (i * 16 + iota16) * 2
        s16 = src_v[pl.ds(i * 16, 16)] * 2
        g0 = plsc.load_gather(table_v, [s16])
        g1 = plsc.load_gather(table_v, [s16 + 1])
        plsc.addupdate_scatter(vals_v, [f16], g0)
        plsc.addupdate_scatter(vals_v, [f16 + 1], g1)
        return carry

    lax.fori_loop(0, EPW // 16, step, 0)

    def sc_step(j, carry):
        pltpu.sync_copy(vals_v.at[pl.ds(j * CH, CH)],
                        agg_sh.at[dst_v.at[j]], add=True)
        return carry

    lax.fori_loop(0, NCH, sc_step, 0)
    plsc.subcore_barrier()

    pltpu.sync_copy(agg_sh.at[pl.ds(sid * RPF, RPF)],
                    out_hbm.at[cid, pl.ds(sid * RPF, RPF)])


@functools.cache
def _sc_agg_kernel():
    return pl.kernel(
        _sc_body,
        out_type=jax.ShapeDtypeStruct((NC, AGF), jnp.float32),
        mesh=plsc.VectorSubcoreMesh(core_axis_name="c", subcore_axis_name="s",
                                    num_cores=NC, num_subcores=NS),
        compiler_params=pltpu.CompilerParams(needs_layout_passes=False),
        scratch_types=[
            pltpu.VMEM((2 * N,), jnp.float32),
            pltpu.VMEM((EPW,), jnp.int32),
            pltpu.VMEM((NCH, CH), jnp.int32),
            pltpu.VMEM((VPW,), jnp.float32),
            pltpu.VMEM_SHARED((AGF,), jnp.float32),
        ],
    )


def kernel(x, edge_index, edge_attr, W_node, b_node, W_msg, b_msg):
    src = edge_index[0].astype(jnp.int32)
    dst = edge_index[1].astype(jnp.int32)

    wc = jnp.concatenate([W_node, W_msg[:D]], axis=1)
    bc = jnp.concatenate([b_node, jnp.zeros((2,), jnp.float32)])[None]
    proj = _node_proj(x, wc, bc)
    node_feat = proj[:, :2]
    node_msg = proj[:, 2:]

    ea_pad = jnp.pad(edge_attr, ((0, E_PAD - E), (0, 0)))
    el = _edge_proj(ea_pad, W_msg[D:], b_msg[None])

    src_pad = jnp.pad(src, (0, E_PAD - E)).reshape(NW, EPW)
    dst_pad = jnp.pad(dst, (0, E_PAD - E), constant_values=N)
    dst2 = jnp.stack([dst_pad * 2, dst_pad * 2 + 1],
                     axis=-1).reshape(NW, NCH, CH)
    zeros_flat = jnp.zeros((AGF,), jnp.float32)

    parts = _sc_agg_kernel()(jnp.reshape(node_msg, (2 * N,)), src_pad, dst2,
                             el.reshape(NW, VPW), zeros_flat)
    parts = parts.reshape(NC, N_A, 2)
    return _finish(node_feat, parts[0, :N], parts[1, :N])


# speedup vs baseline: 3.4314x; 1.3176x over previous
import functools

import jax
import jax.numpy as jnp
from jax import lax
from jax.experimental import pallas as pl
from jax.experimental.pallas import tpu as pltpu
from jax.experimental.pallas import tpu_sc as plsc

N = 10000
D = 256
DE = 16
E = 160000

NC = 2
NS = 16
NW = NC * NS
EPW = 5120
E_PAD = NW * EPW
CH = 128
NCH = EPW // CH
VPW = 2 * EPW
N_A = 10240
RPT = N_A // NS
UNROLL = CH // 16


def _proj_body(x_ref, w_ref, b_ref, o_ref):
    o_ref[...] = jnp.dot(x_ref[...], w_ref[...],
                         preferred_element_type=jnp.float32) + b_ref[...]


def _node_proj(x, wc, bc):
    return pl.pallas_call(
        _proj_body,
        grid=(5,),
        in_specs=[
            pl.BlockSpec((2000, D), lambda i: (i, 0)),
            pl.BlockSpec((D, 4), lambda i: (0, 0)),
            pl.BlockSpec((1, 4), lambda i: (0, 0)),
        ],
        out_specs=pl.BlockSpec((2000, 4), lambda i: (i, 0)),
        out_shape=jax.ShapeDtypeStruct((N, 4), jnp.float32),
    )(x, wc, bc)


def _edge_proj(ea, we, bm):
    return pl.pallas_call(
        _proj_body,
        grid=(10,),
        in_specs=[
            pl.BlockSpec((E // 10, DE), lambda i: (i, 0)),
            pl.BlockSpec((DE, 2), lambda i: (0, 0)),
            pl.BlockSpec((1, 2), lambda i: (0, 0)),
        ],
        out_specs=pl.BlockSpec((E // 10, 2), lambda i: (i, 0)),
        out_shape=jax.ShapeDtypeStruct((E, 2), jnp.float32),
    )(ea, we, bm)


def _finish_body(nf0_ref, nf1_ref, p_ref, o0_ref, o1_ref):
    p = p_ref[...]
    z0 = nf0_ref[...] + p[0, 0] + p[1, 0]
    z1 = nf1_ref[...] + p[0, 1] + p[1, 1]
    m = jnp.maximum(z0, z1)
    l = m + jnp.log(jnp.exp(z0 - m) + jnp.exp(z1 - m))
    o0_ref[...] = z0 - l
    o1_ref[...] = z1 - l


def _finish(nf0, nf1, parts):
    return pl.pallas_call(
        _finish_body,
        out_shape=[jax.ShapeDtypeStruct((N,), jnp.float32)] * 2,
    )(nf0, nf1, parts)


def _sc_body(msg_hbm, src_hbm, dst_hbm, el_hbm, z_hbm, out_hbm,
             table_v, src_v, dst_v, el_v, vals0_v, vals1_v,
             agg0_sh, agg1_sh, sem_in, sem_s):
    cid = lax.axis_index("c")
    sid = lax.axis_index("s")
    wid = cid * NS + sid

    cps = [
        pltpu.async_copy(z_hbm.at[pl.ds(sid * RPT, RPT)],
                         agg0_sh.at[pl.ds(sid * RPT, RPT)], sem_in),
        pltpu.async_copy(z_hbm.at[pl.ds(sid * RPT, RPT)],
                         agg1_sh.at[pl.ds(sid * RPT, RPT)], sem_in),
        pltpu.async_copy(msg_hbm, table_v, sem_in),
        pltpu.async_copy(src_hbm.at[wid], src_v, sem_in),
        pltpu.async_copy(dst_hbm.at[wid], dst_v, sem_in),
        pltpu.async_copy(el_hbm.at[wid], el_v, sem_in),
    ]
    for c in cps:
        c.wait()
    plsc.subcore_barrier()

    iota16 = lax.iota(jnp.int32, 16)

    def chunk(j, carry):
        for k in range(UNROLL):
            i = j * UNROLL + k
            f2 = (i * 16 + iota16) * 2
            s2 = src_v[pl.ds(i * 16, 16)] * 2
            g0 = plsc.load_gather(table_v, [s2])
            g1 = plsc.load_gather(table_v, [s2 + 1])
            e0 = plsc.load_gather(el_v, [f2])
            e1 = plsc.load_gather(el_v, [f2 + 1])
            vals0_v[pl.ds(i * 16, 16)] = g0 + e0
            vals1_v[pl.ds(i * 16, 16)] = g1 + e1
        pltpu.async_copy(vals0_v.at[pl.ds(j * CH, CH)],
                         agg0_sh.at[dst_v.at[j]], sem_s, add=True)
        pltpu.async_copy(vals1_v.at[pl.ds(j * CH, CH)],
                         agg1_sh.at[dst_v.at[j]], sem_s, add=True)
        return carry

    lax.fori_loop(0, NCH, chunk, 0)

    pltpu.make_async_copy(el_hbm.at[wid], el_v, sem_s).wait()
    plsc.subcore_barrier()

    pltpu.sync_copy(agg0_sh.at[pl.ds(sid * RPT, RPT)],
                    out_hbm.at[cid, 0, pl.ds(sid * RPT, RPT)])
    pltpu.sync_copy(agg1_sh.at[pl.ds(sid * RPT, RPT)],
                    out_hbm.at[cid, 1, pl.ds(sid * RPT, RPT)])


@functools.cache
def _sc_agg_kernel():
    return pl.kernel(
        _sc_body,
        out_type=jax.ShapeDtypeStruct((NC, 2, N_A), jnp.float32),
        mesh=plsc.VectorSubcoreMesh(core_axis_name="c", subcore_axis_name="s",
                                    num_cores=NC, num_subcores=NS),
        compiler_params=pltpu.CompilerParams(needs_layout_passes=False),
        scratch_types=[
            pltpu.VMEM((2 * N,), jnp.float32),
            pltpu.VMEM((EPW,), jnp.int32),
            pltpu.VMEM((NCH, CH), jnp.int32),
            pltpu.VMEM((VPW,), jnp.float32),
            pltpu.VMEM((EPW,), jnp.float32),
            pltpu.VMEM((EPW,), jnp.float32),
            pltpu.VMEM_SHARED((N_A,), jnp.float32),
            pltpu.VMEM_SHARED((N_A,), jnp.float32),
            pltpu.SemaphoreType.DMA,
            pltpu.SemaphoreType.DMA,
        ],
    )


def kernel(x, edge_index, edge_attr, W_node, b_node, W_msg, b_msg):
    src = edge_index[0].astype(jnp.int32)
    dst = edge_index[1].astype(jnp.int32)

    wc = jnp.concatenate([W_node, W_msg[:D]], axis=1)
    bc = jnp.concatenate([b_node, jnp.zeros((2,), jnp.float32)])[None]
    proj = _node_proj(x, wc, bc)

    el = _edge_proj(edge_attr, W_msg[D:], b_msg[None])
    el_pad = jnp.pad(el, ((0, E_PAD - E), (0, 0)))

    src_pad = jnp.pad(src, (0, E_PAD - E)).reshape(NW, EPW)
    dst_pad = jnp.pad(dst, (0, E_PAD - E),
                      constant_values=N).reshape(NW, NCH, CH)
    zeros_na = jnp.zeros((N_A,), jnp.float32)

    parts = _sc_agg_kernel()(jnp.reshape(proj[:, 2:], (2 * N,)), src_pad,
                             dst_pad, el_pad.reshape(NW, VPW), zeros_na)
    o0, o1 = _finish(proj[:, 0], proj[:, 1], parts[:, :, :N])
    return jnp.stack([o0, o1], axis=1)


# speedup vs baseline: 12.6548x; 3.6879x over previous
import functools

import jax
import jax.numpy as jnp
from jax import lax
from jax.experimental import pallas as pl
from jax.experimental.pallas import tpu as pltpu
from jax.experimental.pallas import tpu_sc as plsc

N = 10000
D = 256
DE = 16
E = 160000

NC = 2
NS = 16
NW = NC * NS
EPW = 5120
E_PAD = NW * EPW
CH = 128
NCH = EPW // CH
N_A = 10240
RPT = N_A // NS
UNROLL = CH // 16


def _proj_body(x_ref, w_ref, b_ref, o_ref):
    o_ref[...] = jnp.dot(x_ref[...], w_ref[...],
                         preferred_element_type=jnp.float32) + b_ref[...]


def _node_proj(x, wc, bc):
    return pl.pallas_call(
        _proj_body,
        grid=(5,),
        in_specs=[
            pl.BlockSpec((2000, D), lambda i: (i, 0)),
            pl.BlockSpec((D, 4), lambda i: (0, 0)),
            pl.BlockSpec((1, 4), lambda i: (0, 0)),
        ],
        out_specs=pl.BlockSpec((2000, 4), lambda i: (i, 0)),
        out_shape=jax.ShapeDtypeStruct((N, 4), jnp.float32),
    )(x, wc, bc)


def _edge_proj_t(ea_t, we_t, bm_t):
    return pl.pallas_call(
        _proj_body,
        grid=(10,),
        in_specs=[
            pl.BlockSpec((2, DE), lambda i: (0, 0)),
            pl.BlockSpec((DE, E // 10), lambda i: (0, i)),
            pl.BlockSpec((2, 1), lambda i: (0, 0)),
        ],
        out_specs=pl.BlockSpec((2, E // 10), lambda i: (0, i)),
        out_shape=jax.ShapeDtypeStruct((2, E), jnp.float32),
    )(we_t, ea_t, bm_t)


def _finish_body(nf0_ref, nf1_ref, p_ref, o0_ref, o1_ref):
    p = p_ref[...]
    z0 = nf0_ref[...] + p[0, 0] + p[1, 0]
    z1 = nf1_ref[...] + p[0, 1] + p[1, 1]
    m = jnp.maximum(z0, z1)
    l = m + jnp.log(jnp.exp(z0 - m) + jnp.exp(z1 - m))
    o0_ref[...] = z0 - l
    o1_ref[...] = z1 - l


def _finish(nf0, nf1, parts):
    return pl.pallas_call(
        _finish_body,
        out_shape=[jax.ShapeDtypeStruct((N,), jnp.float32)] * 2,
    )(nf0, nf1, parts)


def _sc_body(msg_hbm, src_hbm, dst_hbm, el0_hbm, el1_hbm, z_hbm, out_hbm,
             table_v, src_v, dst_v, vals0_v, vals1_v,
             agg0_sh, agg1_sh, sem_in, sem_s):
    cid = lax.axis_index("c")
    sid = lax.axis_index("s")
    wid = cid * NS + sid

    cps = [
        pltpu.async_copy(z_hbm.at[pl.ds(sid * RPT, RPT)],
                         agg0_sh.at[pl.ds(sid * RPT, RPT)], sem_in),
        pltpu.async_copy(z_hbm.at[pl.ds(sid * RPT, RPT)],
                         agg1_sh.at[pl.ds(sid * RPT, RPT)], sem_in),
        pltpu.async_copy(msg_hbm, table_v, sem_in),
        pltpu.async_copy(src_hbm.at[wid], src_v, sem_in),
        pltpu.async_copy(dst_hbm.at[wid], dst_v, sem_in),
        pltpu.async_copy(el0_hbm.at[wid], vals0_v, sem_in),
        pltpu.async_copy(el1_hbm.at[wid], vals1_v, sem_in),
    ]
    for c in cps:
        c.wait()
    plsc.subcore_barrier()

    def chunk(j, carry):
        for k in range(UNROLL):
            i = j * UNROLL + k
            s2 = src_v[pl.ds(i * 16, 16)] * 2
            g0 = plsc.load_gather(table_v, [s2])
            g1 = plsc.load_gather(table_v, [s2 + 1])
            vals0_v[pl.ds(i * 16, 16)] = vals0_v[pl.ds(i * 16, 16)] + g0
            vals1_v[pl.ds(i * 16, 16)] = vals1_v[pl.ds(i * 16, 16)] + g1
        pltpu.async_copy(vals0_v.at[pl.ds(j * CH, CH)],
                         agg0_sh.at[dst_v.at[j]], sem_s, add=True)
        pltpu.async_copy(vals1_v.at[pl.ds(j * CH, CH)],
                         agg1_sh.at[dst_v.at[j]], sem_s, add=True)
        return carry

    lax.fori_loop(0, NCH, chunk, 0)

    pltpu.make_async_copy(el0_hbm.at[wid], vals0_v, sem_s).wait()
    pltpu.make_async_copy(el1_hbm.at[wid], vals1_v, sem_s).wait()
    plsc.subcore_barrier()

    pltpu.sync_copy(agg0_sh.at[pl.ds(sid * RPT, RPT)],
                    out_hbm.at[cid, 0, pl.ds(sid * RPT, RPT)])
    pltpu.sync_copy(agg1_sh.at[pl.ds(sid * RPT, RPT)],
                    out_hbm.at[cid, 1, pl.ds(sid * RPT, RPT)])


@functools.cache
def _sc_agg_kernel():
    return pl.kernel(
        _sc_body,
        out_type=jax.ShapeDtypeStruct((NC, 2, N_A), jnp.float32),
        mesh=plsc.VectorSubcoreMesh(core_axis_name="c", subcore_axis_name="s",
                                    num_cores=NC, num_subcores=NS),
        compiler_params=pltpu.CompilerParams(needs_layout_passes=False),
        scratch_types=[
            pltpu.VMEM((2 * N,), jnp.float32),
            pltpu.VMEM((EPW,), jnp.int32),
            pltpu.VMEM((NCH, CH), jnp.int32),
            pltpu.VMEM((EPW,), jnp.float32),
            pltpu.VMEM((EPW,), jnp.float32),
            pltpu.VMEM_SHARED((N_A,), jnp.float32),
            pltpu.VMEM_SHARED((N_A,), jnp.float32),
            pltpu.SemaphoreType.DMA,
            pltpu.SemaphoreType.DMA,
        ],
    )


def kernel(x, edge_index, edge_attr, W_node, b_node, W_msg, b_msg):
    src = edge_index[0].astype(jnp.int32)
    dst = edge_index[1].astype(jnp.int32)

    wc = jnp.concatenate([W_node, W_msg[:D]], axis=1)
    bc = jnp.concatenate([b_node, jnp.zeros((2,), jnp.float32)])[None]
    proj = _node_proj(x, wc, bc)

    el_t = _edge_proj_t(edge_attr.T, W_msg[D:].T, b_msg[:, None])
    el_t = jnp.pad(el_t, ((0, 0), (0, E_PAD - E)))

    src_pad = jnp.pad(src, (0, E_PAD - E)).reshape(NW, EPW)
    dst_pad = jnp.pad(dst, (0, E_PAD - E),
                      constant_values=N).reshape(NW, NCH, CH)
    zeros_na = jnp.zeros((N_A,), jnp.float32)

    parts = _sc_agg_kernel()(jnp.reshape(proj[:, 2:], (2 * N,)), src_pad,
                             dst_pad, el_t[0].reshape(NW, EPW),
                             el_t[1].reshape(NW, EPW), zeros_na)
    o0, o1 = _finish(proj[:, 0], proj[:, 1], parts[:, :, :N])
    return jnp.stack([o0, o1], axis=1)


# speedup vs baseline: 15.3157x; 1.2103x over previous
import functools

import jax
import jax.numpy as jnp
from jax import lax
from jax.experimental import pallas as pl
from jax.experimental.pallas import tpu as pltpu
from jax.experimental.pallas import tpu_sc as plsc

N = 10000
D = 256
DE = 16
E = 160000

NC = 2
NS = 16
NW = NC * NS
EPW = 5120
E_PAD = NW * EPW
CH = 128
NCH = EPW // CH
N_A = 10240
RPT = N_A // NS
TW = 2 * N_A
UNROLL = CH // 16


def _proj_body(x_ref, w_ref, b_ref, o_ref):
    o_ref[...] = jnp.dot(x_ref[...], w_ref[...],
                         preferred_element_type=jnp.float32) + b_ref[...]


def _node_proj(x, wc, bc):
    return pl.pallas_call(
        _proj_body,
        grid=(5,),
        in_specs=[
            pl.BlockSpec((2000, D), lambda i: (i, 0)),
            pl.BlockSpec((D, 4), lambda i: (0, 0)),
            pl.BlockSpec((1, 4), lambda i: (0, 0)),
        ],
        out_specs=pl.BlockSpec((2000, 4), lambda i: (i, 0)),
        out_shape=jax.ShapeDtypeStruct((N, 4), jnp.float32),
    )(x, wc, bc)


def _edge_proj_t(ea_t, we_t, bm_t):
    return pl.pallas_call(
        _proj_body,
        grid=(10,),
        in_specs=[
            pl.BlockSpec((2, DE), lambda i: (0, 0)),
            pl.BlockSpec((DE, E // 10), lambda i: (0, i)),
            pl.BlockSpec((2, 1), lambda i: (0, 0)),
        ],
        out_specs=pl.BlockSpec((2, E // 10), lambda i: (0, i)),
        out_shape=jax.ShapeDtypeStruct((2, E), jnp.float32),
    )(we_t, ea_t, bm_t)


def _finish_body(nf0_ref, nf1_ref, p_ref, o_ref):
    p = p_ref[...]
    z0 = nf0_ref[...] + p[0, 0] + p[1, 0]
    z1 = nf1_ref[...] + p[0, 1] + p[1, 1]
    m = jnp.maximum(z0, z1)
    l = m + jnp.log(jnp.exp(z0 - m) + jnp.exp(z1 - m))
    o_ref[0, :] = z0 - l
    o_ref[1, :] = z1 - l


def _finish(nf0, nf1, parts):
    return pl.pallas_call(
        _finish_body,
        out_shape=jax.ShapeDtypeStruct((2, N), jnp.float32),
    )(nf0, nf1, parts)


def _sc_body(msg_hbm, ei_hbm, el_hbm, z_hbm, out_hbm,
             table_v, src_v, dst_v, vals0_v, vals1_v,
             agg0_sh, agg1_sh, sem_in, sem_s):
    cid = lax.axis_index("c")
    sid = lax.axis_index("s")
    wid = cid * NS + sid

    cps = [
        pltpu.async_copy(z_hbm.at[pl.ds(sid * RPT, RPT)],
                         agg0_sh.at[pl.ds(sid * RPT, RPT)], sem_in),
        pltpu.async_copy(z_hbm.at[pl.ds(sid * RPT, RPT)],
                         agg1_sh.at[pl.ds(sid * RPT, RPT)], sem_in),
        pltpu.async_copy(msg_hbm, table_v.at[pl.ds(0, 2 * N)], sem_in),
        pltpu.async_copy(ei_hbm.at[0, wid], src_v, sem_in),
        pltpu.async_copy(ei_hbm.at[1, wid], dst_v, sem_in),
        pltpu.async_copy(el_hbm.at[0, pl.ds(wid * EPW, EPW)], vals0_v, sem_in),
        pltpu.async_copy(el_hbm.at[1, pl.ds(wid * EPW, EPW)], vals1_v, sem_in),
    ]
    for c in cps:
        c.wait()
    plsc.subcore_barrier()

    def chunk(j, carry):
        for k in range(UNROLL):
            i = j * UNROLL + k
            s2 = src_v[j, pl.ds(k * 16, 16)] * 2
            g0 = plsc.load_gather(table_v, [s2])
            g1 = plsc.load_gather(table_v, [s2 + 1])
            vals0_v[pl.ds(i * 16, 16)] = vals0_v[pl.ds(i * 16, 16)] + g0
            vals1_v[pl.ds(i * 16, 16)] = vals1_v[pl.ds(i * 16, 16)] + g1
        pltpu.async_copy(vals0_v.at[pl.ds(j * CH, CH)],
                         agg0_sh.at[dst_v.at[j]], sem_s, add=True)
        pltpu.async_copy(vals1_v.at[pl.ds(j * CH, CH)],
                         agg1_sh.at[dst_v.at[j]], sem_s, add=True)
        return carry

    lax.fori_loop(0, NCH, chunk, 0)

    pltpu.make_async_copy(el_hbm.at[0, pl.ds(wid * EPW, EPW)],
                          vals0_v, sem_s).wait()
    pltpu.make_async_copy(el_hbm.at[1, pl.ds(wid * EPW, EPW)],
                          vals1_v, sem_s).wait()
    plsc.subcore_barrier()

    pltpu.sync_copy(agg0_sh.at[pl.ds(sid * RPT, RPT)],
                    out_hbm.at[cid, 0, pl.ds(sid * RPT, RPT)])
    pltpu.sync_copy(agg1_sh.at[pl.ds(sid * RPT, RPT)],
                    out_hbm.at[cid, 1, pl.ds(sid * RPT, RPT)])


@functools.cache
def _sc_agg_kernel():
    return pl.kernel(
        _sc_body,
        out_type=jax.ShapeDtypeStruct((NC, 2, N_A), jnp.float32),
        mesh=plsc.VectorSubcoreMesh(core_axis_name="c", subcore_axis_name="s",
                                    num_cores=NC, num_subcores=NS),
        compiler_params=pltpu.CompilerParams(needs_layout_passes=False),
        scratch_types=[
            pltpu.VMEM((TW,), jnp.float32),
            pltpu.VMEM((NCH, CH), jnp.int32),
            pltpu.VMEM((NCH, CH), jnp.int32),
            pltpu.VMEM((EPW,), jnp.float32),
            pltpu.VMEM((EPW,), jnp.float32),
            pltpu.VMEM_SHARED((N_A,), jnp.float32),
            pltpu.VMEM_SHARED((N_A,), jnp.float32),
            pltpu.SemaphoreType.DMA,
            pltpu.SemaphoreType.DMA,
        ],
    )


def kernel(x, edge_index, edge_attr, W_node, b_node, W_msg, b_msg):
    ei = edge_index.astype(jnp.int32)
    ei_pad = jnp.pad(ei, ((0, 0), (0, E_PAD - E)),
                     constant_values=N).reshape(2, NW, NCH, CH)

    wc = jnp.concatenate([W_node, W_msg[:D]], axis=1)
    bc = jnp.concatenate([b_node, jnp.zeros((2,), jnp.float32)])[None]
    proj = _node_proj(x, wc, bc)

    el_t = _edge_proj_t(edge_attr.T, W_msg[D:].T, b_msg[:, None])
    el_t = jnp.pad(el_t, ((0, 0), (0, E_PAD - E)))

    zeros_na = jnp.zeros((N_A,), jnp.float32)

    parts = _sc_agg_kernel()(jnp.reshape(proj[:, 2:], (2 * N,)),
                             ei_pad, el_t, zeros_na)
    out_t = _finish(proj[:, 0], proj[:, 1], parts[:, :, :N])
    return out_t.T


# speedup vs baseline: 18.6094x; 1.2151x over previous
import functools

import jax
import jax.numpy as jnp
from jax import lax
from jax.experimental import pallas as pl
from jax.experimental.pallas import tpu as pltpu
from jax.experimental.pallas import tpu_sc as plsc

N = 10000
D = 256
DE = 16
E = 160000

NC = 2
NS = 16
NW = NC * NS
EPW = 5120
E_PAD = NW * EPW
CH = 128
NCH = EPW // CH
N_A = 10240
RPT = N_A // NS
TW = 2 * N_A
UNROLL = CH // 16


def _proj_t_body(w_ref, x_ref, b_ref, o_ref):
    o_ref[...] = lax.dot_general(
        w_ref[...], x_ref[...], (((0,), (1,)), ((), ())),
        preferred_element_type=jnp.float32) + b_ref[...]


def _node_proj_t(x, wc, bc_t):
    return pl.pallas_call(
        _proj_t_body,
        out_shape=jax.ShapeDtypeStruct((4, N), jnp.float32),
    )(wc, x, bc_t)


def _edge_proj_t_body(w_ref, ea_ref, b_ref, o_ref):
    o_ref[...] = jnp.dot(w_ref[...], ea_ref[...],
                         preferred_element_type=jnp.float32) + b_ref[...]


def _edge_proj_t(ea_t, we_t, bm_t):
    return pl.pallas_call(
        _edge_proj_t_body,
        grid=(10,),
        in_specs=[
            pl.BlockSpec((2, DE), lambda i: (0, 0)),
            pl.BlockSpec((DE, E_PAD // 10), lambda i: (0, i)),
            pl.BlockSpec((2, 1), lambda i: (0, 0)),
        ],
        out_specs=pl.BlockSpec((2, E_PAD // 10), lambda i: (0, i)),
        out_shape=jax.ShapeDtypeStruct((2, E_PAD), jnp.float32),
    )(we_t, ea_t, bm_t)


def _finish_body(p4_ref, p_ref, o_ref):
    p = p_ref[...]
    z0 = p4_ref[pl.ds(0, N)] + p[0, 0, :N] + p[1, 0, :N]
    z1 = p4_ref[pl.ds(N, N)] + p[0, 1, :N] + p[1, 1, :N]
    m = jnp.maximum(z0, z1)
    l = m + jnp.log(jnp.exp(z0 - m) + jnp.exp(z1 - m))
    o_ref[0, :] = z0 - l
    o_ref[1, :] = z1 - l


def _finish(proj_t, parts):
    return pl.pallas_call(
        _finish_body,
        out_shape=jax.ShapeDtypeStruct((2, N), jnp.float32),
    )(proj_t, parts)


def _sc_body(proj_hbm, ei_hbm, el_hbm, z_hbm, out_hbm,
             table_v, src_v, dst_v, vals0_v, vals1_v,
             agg0_sh, agg1_sh, sem_in, sem_s):
    cid = lax.axis_index("c")
    sid = lax.axis_index("s")
    wid = cid * NS + sid

    cps = [
        pltpu.async_copy(z_hbm.at[pl.ds(sid * RPT, RPT)],
                         agg0_sh.at[pl.ds(sid * RPT, RPT)], sem_in),
        pltpu.async_copy(z_hbm.at[pl.ds(sid * RPT, RPT)],
                         agg1_sh.at[pl.ds(sid * RPT, RPT)], sem_in),
        pltpu.async_copy(proj_hbm.at[pl.ds(2 * N, N)],
                         table_v.at[pl.ds(0, N)], sem_in),
        pltpu.async_copy(proj_hbm.at[pl.ds(3 * N, N)],
                         table_v.at[pl.ds(N_A, N)], sem_in),
        pltpu.async_copy(ei_hbm.at[0, wid], src_v, sem_in),
        pltpu.async_copy(ei_hbm.at[1, wid], dst_v, sem_in),
        pltpu.async_copy(el_hbm.at[0, pl.ds(wid * EPW, EPW)], vals0_v, sem_in),
        pltpu.async_copy(el_hbm.at[1, pl.ds(wid * EPW, EPW)], vals1_v, sem_in),
    ]
    for c in cps:
        c.wait()
    plsc.subcore_barrier()

    def chunk(j, carry):
        for k in range(UNROLL):
            s16 = src_v[j, pl.ds(k * 16, 16)]
            g0 = plsc.load_gather(table_v, [s16])
            g1 = plsc.load_gather(table_v, [s16 + N_A])
            i = j * UNROLL + k
            vals0_v[pl.ds(i * 16, 16)] = vals0_v[pl.ds(i * 16, 16)] + g0
            vals1_v[pl.ds(i * 16, 16)] = vals1_v[pl.ds(i * 16, 16)] + g1
        pltpu.async_copy(vals0_v.at[pl.ds(j * CH, CH)],
                         agg0_sh.at[dst_v.at[j]], sem_s, add=True)
        pltpu.async_copy(vals1_v.at[pl.ds(j * CH, CH)],
                         agg1_sh.at[dst_v.at[j]], sem_s, add=True)
        return carry

    lax.fori_loop(0, NCH, chunk, 0)

    pltpu.make_async_copy(el_hbm.at[0, pl.ds(wid * EPW, EPW)],
                          vals0_v, sem_s).wait()
    pltpu.make_async_copy(el_hbm.at[1, pl.ds(wid * EPW, EPW)],
                          vals1_v, sem_s).wait()
    plsc.subcore_barrier()

    pltpu.sync_copy(agg0_sh.at[pl.ds(sid * RPT, RPT)],
                    out_hbm.at[cid, 0, pl.ds(sid * RPT, RPT)])
    pltpu.sync_copy(agg1_sh.at[pl.ds(sid * RPT, RPT)],
                    out_hbm.at[cid, 1, pl.ds(sid * RPT, RPT)])


@functools.cache
def _sc_agg_kernel():
    return pl.kernel(
        _sc_body,
        out_type=jax.ShapeDtypeStruct((NC, 2, N_A), jnp.float32),
        mesh=plsc.VectorSubcoreMesh(core_axis_name="c", subcore_axis_name="s",
                                    num_cores=NC, num_subcores=NS),
        compiler_params=pltpu.CompilerParams(needs_layout_passes=False),
        scratch_types=[
            pltpu.VMEM((TW,), jnp.float32),
            pltpu.VMEM((NCH, CH), jnp.int32),
            pltpu.VMEM((NCH, CH), jnp.int32),
            pltpu.VMEM((EPW,), jnp.float32),
            pltpu.VMEM((EPW,), jnp.float32),
            pltpu.VMEM_SHARED((N_A,), jnp.float32),
            pltpu.VMEM_SHARED((N_A,), jnp.float32),
            pltpu.SemaphoreType.DMA,
            pltpu.SemaphoreType.DMA,
        ],
    )


def kernel(x, edge_index, edge_attr, W_node, b_node, W_msg, b_msg):
    ei = edge_index.astype(jnp.int32)
    ei_pad = jnp.pad(ei, ((0, 0), (0, E_PAD - E)),
                     constant_values=N).reshape(2, NW, NCH, CH)

    wc = jnp.concatenate([W_node, W_msg[:D]], axis=1)
    bc_t = jnp.concatenate([b_node, jnp.zeros((2,), jnp.float32)])[:, None]
    proj_flat = jnp.reshape(_node_proj_t(x, wc, bc_t), (4 * N,))

    el_t = _edge_proj_t(edge_attr.T, W_msg[D:].T, b_msg[:, None])

    zeros_na = jnp.zeros((N_A,), jnp.float32)

    parts = _sc_agg_kernel()(proj_flat, ei_pad, el_t, zeros_na)
    return _finish(proj_flat, parts).T
